# baseline (speedup 1.0000x reference)
import jax
import jax.numpy as jnp
from jax.experimental import pallas as pl
from jax.experimental.pallas import tpu as pltpu

_EMBED_DIM = 1024
_BATCH = 4


_CHUNKS = [(0, 16), (16, 16), (32, 32), (64, 64), (128, 128)]


def _copy_body(table_hbm, out_hbm, vmem, sem_in, sem_out):
    for i, (lo, sz) in enumerate(_CHUNKS):
        pltpu.make_async_copy(
            table_hbm.at[pl.ds(lo, sz)],
            vmem.at[pl.ds(lo, sz)],
            sem_in.at[i],
        ).start()
    for i, (lo, sz) in enumerate(_CHUNKS):
        pltpu.make_async_copy(
            table_hbm.at[pl.ds(lo, sz)],
            vmem.at[pl.ds(lo, sz)],
            sem_in.at[i],
        ).wait()
        for b in range(_BATCH):
            pltpu.make_async_copy(
                vmem.at[pl.ds(lo, sz)],
                out_hbm.at[b, pl.ds(lo, sz)],
                sem_out,
            ).start()
    for lo, sz in _CHUNKS:
        for b in range(_BATCH):
            pltpu.make_async_copy(
                vmem.at[pl.ds(lo, sz)],
                out_hbm.at[b, pl.ds(lo, sz)],
                sem_out,
            ).wait()


def kernel(num_bars, batch_size, embedding_weight):
    n = embedding_weight.shape[0]
    return pl.pallas_call(
        _copy_body,
        in_specs=[pl.BlockSpec(memory_space=pltpu.HBM)],
        out_specs=pl.BlockSpec(memory_space=pltpu.HBM),
        out_shape=jax.ShapeDtypeStruct((_BATCH, n, _EMBED_DIM), jnp.float32),
        scratch_shapes=[
            pltpu.VMEM((n, _EMBED_DIM), jnp.float32),
            pltpu.SemaphoreType.DMA((len(_CHUNKS),)),
            pltpu.SemaphoreType.DMA,
        ],
    )(embedding_weight)
